# baseline (speedup 1.0000x reference)
import functools

import jax
import jax.numpy as jnp
from jax import lax
from jax.experimental import pallas as pl
from jax.experimental.pallas import tpu as pltpu
from jax.experimental.pallas import tpu_sc as plsc

_EPS = 1e-08
_B, _N, _M = 4, 2304, 2304
_NW = 32
_L = 16


def _tc_body(src_ref, tgt_ref, omin_ref, oarg_ref, *, tn, m):
    q = src_ref[0]
    sx = q[:, 0:1]
    sy = q[:, 1:2]
    t = tgt_ref[0]
    tx = t[0:1, :]
    ty = t[1:2, :]
    dx = jnp.abs(sx - tx) + _EPS
    dy = jnp.abs(sy - ty) + _EPS
    s = dx * dx + dy * dy
    mn = jnp.min(s, axis=1, keepdims=True)
    idx = lax.broadcasted_iota(jnp.int32, (tn, m), 1)
    am = jnp.min(jnp.where(s <= mn, idx, m), axis=1, keepdims=True)
    omin_ref[0] = mn
    oarg_ref[0] = am


def _pairwise_min_tc(src, tgt2, *, row0, rows, tn):
    b = src.shape[0]
    m = tgt2.shape[2]
    grid = (b, rows // tn)
    r0 = row0 // tn
    mn, am = pl.pallas_call(
        functools.partial(_tc_body, tn=tn, m=m),
        grid=grid,
        in_specs=[
            pl.BlockSpec((1, tn, 2), lambda bi, i: (bi, r0 + i, 0)),
            pl.BlockSpec((1, 2, m), lambda bi, i: (bi, 0, 0)),
        ],
        out_specs=[
            pl.BlockSpec((1, tn, 1), lambda bi, i: (bi, i, 0)),
            pl.BlockSpec((1, tn, 1), lambda bi, i: (bi, i, 0)),
        ],
        out_shape=[
            jax.ShapeDtypeStruct((b, rows, 1), jnp.float32),
            jax.ShapeDtypeStruct((b, rows, 1), jnp.int32),
        ],
    )(src, tgt2)
    return mn, am


def _sc_body(sxq_hbm, syq_hbm, tfl_hbm, om_hbm, oa_hbm,
             qx_v, qy_v, txy_v, om_v, oa_v, *, k):
    chunk = k // 8
    c = lax.axis_index("c")
    s = lax.axis_index("s")
    wid = s * 2 + c
    b = wid // 8
    cw = wid % 8
    qbase = b * _N + cw * chunk
    obase = b * k + cw * chunk
    pltpu.sync_copy(sxq_hbm.at[pl.ds(qbase, chunk)], qx_v)
    pltpu.sync_copy(syq_hbm.at[pl.ds(qbase, chunk)], qy_v)
    pltpu.sync_copy(tfl_hbm.at[pl.ds(2 * b * _M, 2 * _M)], txy_v)

    for g in range(chunk // _L):
        qx = qx_v[pl.ds(g * _L, _L)]
        qy = qy_v[pl.ds(g * _L, _L)]

        def body(j, carry, qx=qx, qy=qy):
            m, am = carry
            va = txy_v[pl.ds(2 * _L * j, _L)]
            vb = txy_v[pl.ds(2 * _L * j + _L, _L)]
            tb = j * _L
            for u in range(_L):
                h = va if u < _L // 2 else vb
                lx = (2 * u) % _L
                dx = jnp.abs(qx - h[lx]) + _EPS
                dy = jnp.abs(qy - h[lx + 1]) + _EPS
                sq = dx * dx + dy * dy
                ltm = sq < m
                am = jnp.where(ltm, tb + u, am)
                m = jnp.minimum(m, sq)
            return m, am

        m0 = jnp.full((_L,), jnp.inf, jnp.float32)
        am0 = jnp.zeros((_L,), jnp.int32)
        m, am = lax.fori_loop(0, _M // _L, body, (m0, am0))
        om_v[pl.ds(g * _L, _L)] = m
        oa_v[pl.ds(g * _L, _L)] = am

    pltpu.sync_copy(om_v, om_hbm.at[pl.ds(obase, chunk)])
    pltpu.sync_copy(oa_v, oa_hbm.at[pl.ds(obase, chunk)])


def _pairwise_min_sc(sxq, syq, tfl, *, k):
    chunk = k // 8
    mesh = plsc.VectorSubcoreMesh(core_axis_name="c", subcore_axis_name="s")
    run = functools.partial(
        pl.kernel,
        mesh=mesh,
        out_type=[
            jax.ShapeDtypeStruct((_B * k,), jnp.float32),
            jax.ShapeDtypeStruct((_B * k,), jnp.int32),
        ],
        scratch_types=[
            pltpu.VMEM((chunk,), jnp.float32),
            pltpu.VMEM((chunk,), jnp.float32),
            pltpu.VMEM((2 * _M,), jnp.float32),
            pltpu.VMEM((chunk,), jnp.float32),
            pltpu.VMEM((chunk,), jnp.int32),
        ],
    )(functools.partial(_sc_body, k=k))
    return run(sxq, syq, tfl)


_K_SC = 768
_TN = 384


@jax.jit
def kernel(source_uv_warped, target_uv_pred):
    b = source_uv_warped.shape[0]
    src = jnp.reshape(source_uv_warped, (b, -1, 2))
    tgt = jnp.reshape(target_uv_pred, (b, -1, 2))
    n = src.shape[1]
    sxq = src[:, :, 0].reshape(-1)
    syq = src[:, :, 1].reshape(-1)
    tfl = jnp.reshape(target_uv_pred, (-1,))
    mn_sc, am_sc = _pairwise_min_sc(sxq, syq, tfl, k=_K_SC)
    tgt2 = jnp.transpose(tgt, (0, 2, 1))
    nk = n - _K_SC
    mn_tc, am_tc = _pairwise_min_tc(src, tgt2, row0=_K_SC, rows=nk, tn=_TN)
    mn = jnp.concatenate(
        [mn_sc.reshape(b, _K_SC), mn_tc.reshape(b, nk)], axis=1)
    am = jnp.concatenate(
        [am_sc.reshape(b, _K_SC), am_tc.reshape(b, nk)], axis=1)
    return (jnp.sqrt(mn), am)


# speedup vs baseline: 1.1862x; 1.1862x over previous
import functools

import jax
import jax.numpy as jnp
from jax import lax
from jax.experimental import pallas as pl
from jax.experimental.pallas import tpu as pltpu
from jax.experimental.pallas import tpu_sc as plsc

_EPS = 1e-08
_B, _N, _M = 4, 2304, 2304
_NW = 32
_L = 16


def _tc_body(sx_ref, sy_ref, tx_ref, ty_ref, omin_ref, oarg_ref, *, tn, m):
    sx = sx_ref[0]
    sy = sy_ref[0]
    tx = tx_ref[0]
    ty = ty_ref[0]
    dx = jnp.abs(sx - tx) + _EPS
    dy = jnp.abs(sy - ty) + _EPS
    s = dx * dx + dy * dy
    mn = jnp.min(s, axis=1, keepdims=True)
    idx = lax.broadcasted_iota(jnp.int32, (tn, m), 1)
    am = jnp.min(jnp.where(s <= mn, idx, m), axis=1, keepdims=True)
    omin_ref[0] = mn
    oarg_ref[0] = am


def _pairwise_min_tc(sx, sy, tx, ty, *, tn):
    b, rows, _ = sx.shape
    m = tx.shape[2]
    grid = (b, rows // tn)
    src_spec = pl.BlockSpec((1, tn, 1), lambda bi, i: (bi, i, 0))
    tgt_spec = pl.BlockSpec((1, 1, m), lambda bi, i: (bi, 0, 0))
    out_spec = pl.BlockSpec((1, tn, 1), lambda bi, i: (bi, i, 0))
    mn, am = pl.pallas_call(
        functools.partial(_tc_body, tn=tn, m=m),
        grid=grid,
        in_specs=[src_spec, src_spec, tgt_spec, tgt_spec],
        out_specs=[out_spec, out_spec],
        out_shape=[
            jax.ShapeDtypeStruct((b, rows, 1), jnp.float32),
            jax.ShapeDtypeStruct((b, rows, 1), jnp.int32),
        ],
    )(sx, sy, tx, ty)
    return mn, am


def _sc_body(sxq_hbm, syq_hbm, txq_hbm, tyq_hbm, om_hbm, oa_hbm,
             qx_v, qy_v, tx_v, ty_v, om_v, oa_v, *, k):
    chunk = k // 8
    c = lax.axis_index("c")
    s = lax.axis_index("s")
    wid = s * 2 + c
    b = wid // 8
    cw = wid % 8
    qbase = b * _N + cw * chunk
    obase = b * k + cw * chunk
    pltpu.sync_copy(sxq_hbm.at[pl.ds(qbase, chunk)], qx_v)
    pltpu.sync_copy(syq_hbm.at[pl.ds(qbase, chunk)], qy_v)
    pltpu.sync_copy(txq_hbm.at[pl.ds(b * _M, _M)], tx_v)
    pltpu.sync_copy(tyq_hbm.at[pl.ds(b * _M, _M)], ty_v)

    ngroups = chunk // _L
    g = 0
    while g < ngroups:
        ng = 3 if g + 2 < ngroups else (ngroups - g)
        qs = [(qx_v[pl.ds((g + t) * _L, _L)], qy_v[pl.ds((g + t) * _L, _L)])
              for t in range(ng)]

        def body(j, carry, qs=qs, ng=ng):
            ms, ams = list(carry[0]), list(carry[1])
            tvx = tx_v[pl.ds(j * _L, _L)]
            tvy = ty_v[pl.ds(j * _L, _L)]
            tb = j * _L
            for u in range(_L):
                bx = tvx[u]
                by = tvy[u]
                ti = tb + u
                for t in range(ng):
                    qx, qy = qs[t]
                    dx = jnp.abs(qx - bx) + _EPS
                    dy = jnp.abs(qy - by) + _EPS
                    sq = dx * dx + dy * dy
                    ltm = sq < ms[t]
                    ams[t] = jnp.where(ltm, ti, ams[t])
                    ms[t] = jnp.minimum(ms[t], sq)
            return tuple(ms), tuple(ams)

        m0 = tuple(jnp.full((_L,), jnp.inf, jnp.float32) for _ in range(ng))
        am0 = tuple(jnp.zeros((_L,), jnp.int32) for _ in range(ng))
        ms, ams = lax.fori_loop(0, _M // _L, body, (m0, am0))
        for t in range(ng):
            om_v[pl.ds((g + t) * _L, _L)] = ms[t]
            oa_v[pl.ds((g + t) * _L, _L)] = ams[t]
        g += ng

    pltpu.sync_copy(om_v, om_hbm.at[pl.ds(obase, chunk)])
    pltpu.sync_copy(oa_v, oa_hbm.at[pl.ds(obase, chunk)])


def _pairwise_min_sc(sxq, syq, txq, tyq, *, k):
    chunk = k // 8
    mesh = plsc.VectorSubcoreMesh(core_axis_name="c", subcore_axis_name="s")
    run = functools.partial(
        pl.kernel,
        mesh=mesh,
        out_type=[
            jax.ShapeDtypeStruct((_B * k,), jnp.float32),
            jax.ShapeDtypeStruct((_B * k,), jnp.int32),
        ],
        scratch_types=[
            pltpu.VMEM((chunk,), jnp.float32),
            pltpu.VMEM((chunk,), jnp.float32),
            pltpu.VMEM((_M,), jnp.float32),
            pltpu.VMEM((_M,), jnp.float32),
            pltpu.VMEM((chunk,), jnp.float32),
            pltpu.VMEM((chunk,), jnp.int32),
        ],
    )(functools.partial(_sc_body, k=k))
    return run(sxq, syq, txq, tyq)


_K_SC = 768
_TN = 512


@jax.jit
def kernel(source_uv_warped, target_uv_pred):
    b = source_uv_warped.shape[0]
    src = jnp.reshape(source_uv_warped, (b, -1, 2))
    tgt = jnp.reshape(target_uv_pred, (b, -1, 2))
    n = src.shape[1]
    sxq = src[:, :, 0]
    syq = src[:, :, 1]
    txq = tgt[:, :, 0]
    tyq = tgt[:, :, 1]
    mn_sc, am_sc = _pairwise_min_sc(
        sxq.reshape(-1), syq.reshape(-1), txq.reshape(-1), tyq.reshape(-1),
        k=_K_SC)
    nk = n - _K_SC
    mn_tc, am_tc = _pairwise_min_tc(
        sxq[:, _K_SC:, None], syq[:, _K_SC:, None],
        txq[:, None, :], tyq[:, None, :], tn=_TN)
    mn = jnp.concatenate(
        [mn_sc.reshape(b, _K_SC), mn_tc.reshape(b, nk)], axis=1)
    am = jnp.concatenate(
        [am_sc.reshape(b, _K_SC), am_tc.reshape(b, nk)], axis=1)
    return (jnp.sqrt(mn), am)


# speedup vs baseline: 1.1909x; 1.0040x over previous
import functools

import jax
import jax.numpy as jnp
from jax import lax
from jax.experimental import pallas as pl
from jax.experimental.pallas import tpu as pltpu
from jax.experimental.pallas import tpu_sc as plsc

_EPS = 1e-08
_B, _N, _M = 4, 2304, 2304
_NW = 32
_L = 16


def _tc_body(sx_ref, sy_ref, tx_ref, ty_ref, omin_ref, oarg_ref, *, tn, m):
    sx = sx_ref[0]
    sy = sy_ref[0]
    tx = tx_ref[0]
    ty = ty_ref[0]
    dx = jnp.abs(sx - tx) + _EPS
    dy = jnp.abs(sy - ty) + _EPS
    s = dx * dx + dy * dy
    mn = jnp.min(s, axis=1, keepdims=True)
    idx = lax.broadcasted_iota(jnp.int32, (tn, m), 1)
    am = jnp.min(jnp.where(s <= mn, idx, m), axis=1, keepdims=True)
    omin_ref[0] = mn
    oarg_ref[0] = am


def _pairwise_min_tc(sx, sy, tx, ty, *, tn):
    b, rows, _ = sx.shape
    m = tx.shape[2]
    grid = (b, rows // tn)
    src_spec = pl.BlockSpec((1, tn, 1), lambda bi, i: (bi, i, 0))
    tgt_spec = pl.BlockSpec((1, 1, m), lambda bi, i: (bi, 0, 0))
    out_spec = pl.BlockSpec((1, tn, 1), lambda bi, i: (bi, i, 0))
    mn, am = pl.pallas_call(
        functools.partial(_tc_body, tn=tn, m=m),
        grid=grid,
        in_specs=[src_spec, src_spec, tgt_spec, tgt_spec],
        out_specs=[out_spec, out_spec],
        out_shape=[
            jax.ShapeDtypeStruct((b, rows, 1), jnp.float32),
            jax.ShapeDtypeStruct((b, rows, 1), jnp.int32),
        ],
    )(sx, sy, tx, ty)
    return mn, am


def _sc_body(sxq_hbm, syq_hbm, txq_hbm, tyq_hbm, om_hbm, oa_hbm,
             qx_v, qy_v, tx_v, ty_v, om_v, oa_v, *, k):
    chunk = k // 8
    c = lax.axis_index("c")
    s = lax.axis_index("s")
    wid = s * 2 + c
    b = wid // 8
    cw = wid % 8
    qbase = b * _N + cw * chunk
    obase = b * k + cw * chunk
    pltpu.sync_copy(sxq_hbm.at[pl.ds(qbase, chunk)], qx_v)
    pltpu.sync_copy(syq_hbm.at[pl.ds(qbase, chunk)], qy_v)
    pltpu.sync_copy(txq_hbm.at[pl.ds(b * _M, _M)], tx_v)
    pltpu.sync_copy(tyq_hbm.at[pl.ds(b * _M, _M)], ty_v)

    ngroups = chunk // _L
    g = 0
    while g < ngroups:
        ng = 3 if g + 2 < ngroups else (ngroups - g)
        qs = [(qx_v[pl.ds((g + t) * _L, _L)], qy_v[pl.ds((g + t) * _L, _L)])
              for t in range(ng)]

        def body(j, carry, qs=qs, ng=ng):
            ms, ams = list(carry[0]), list(carry[1])
            tvx = tx_v[pl.ds(j * _L, _L)]
            tvy = ty_v[pl.ds(j * _L, _L)]
            tb = j * _L
            for u in range(_L):
                bx = tvx[u]
                by = tvy[u]
                ti = tb + u
                for t in range(ng):
                    qx, qy = qs[t]
                    dx = jnp.abs(qx - bx) + _EPS
                    dy = jnp.abs(qy - by) + _EPS
                    sq = dx * dx + dy * dy
                    ltm = sq < ms[t]
                    ams[t] = jnp.where(ltm, ti, ams[t])
                    ms[t] = jnp.minimum(ms[t], sq)
            return tuple(ms), tuple(ams)

        m0 = tuple(jnp.full((_L,), jnp.inf, jnp.float32) for _ in range(ng))
        am0 = tuple(jnp.zeros((_L,), jnp.int32) for _ in range(ng))
        ms, ams = lax.fori_loop(0, _M // _L, body, (m0, am0))
        for t in range(ng):
            om_v[pl.ds((g + t) * _L, _L)] = ms[t]
            oa_v[pl.ds((g + t) * _L, _L)] = ams[t]
        g += ng

    pltpu.sync_copy(om_v, om_hbm.at[pl.ds(obase, chunk)])
    pltpu.sync_copy(oa_v, oa_hbm.at[pl.ds(obase, chunk)])


def _pairwise_min_sc(sxq, syq, txq, tyq, *, k):
    chunk = k // 8
    mesh = plsc.VectorSubcoreMesh(core_axis_name="c", subcore_axis_name="s")
    run = functools.partial(
        pl.kernel,
        mesh=mesh,
        out_type=[
            jax.ShapeDtypeStruct((_B * k,), jnp.float32),
            jax.ShapeDtypeStruct((_B * k,), jnp.int32),
        ],
        scratch_types=[
            pltpu.VMEM((chunk,), jnp.float32),
            pltpu.VMEM((chunk,), jnp.float32),
            pltpu.VMEM((_M,), jnp.float32),
            pltpu.VMEM((_M,), jnp.float32),
            pltpu.VMEM((chunk,), jnp.float32),
            pltpu.VMEM((chunk,), jnp.int32),
        ],
    )(functools.partial(_sc_body, k=k))
    return run(sxq, syq, txq, tyq)


_K_SC = 768
_TN = 768


@jax.jit
def kernel(source_uv_warped, target_uv_pred):
    b = source_uv_warped.shape[0]
    src = jnp.reshape(source_uv_warped, (b, -1, 2))
    tgt = jnp.reshape(target_uv_pred, (b, -1, 2))
    n = src.shape[1]
    sxq = src[:, :, 0]
    syq = src[:, :, 1]
    txq = tgt[:, :, 0]
    tyq = tgt[:, :, 1]
    mn_sc, am_sc = _pairwise_min_sc(
        sxq.reshape(-1), syq.reshape(-1), txq.reshape(-1), tyq.reshape(-1),
        k=_K_SC)
    nk = n - _K_SC
    mn_tc, am_tc = _pairwise_min_tc(
        sxq[:, _K_SC:, None], syq[:, _K_SC:, None],
        txq[:, None, :], tyq[:, None, :], tn=_TN)
    mn = jnp.concatenate(
        [mn_sc.reshape(b, _K_SC), mn_tc.reshape(b, nk)], axis=1)
    am = jnp.concatenate(
        [am_sc.reshape(b, _K_SC), am_tc.reshape(b, nk)], axis=1)
    return (jnp.sqrt(mn), am)
